# baseline (speedup 1.0000x reference)
import functools

import jax
import jax.numpy as jnp
from jax import lax
from jax.experimental import pallas as pl
from jax.experimental.pallas import tpu as pltpu
from jax.experimental.pallas import tpu_sc as plsc

B = 4096
L = 50
LP = 56
DIM = 128
NLANE = 16


def _gather_sum(idx0, idx1, emb0, emb1):
    nb = idx0.shape[0]
    info = plsc.get_sparse_core_info()
    nw = info.num_cores * info.num_subcores
    assert nb % nw == 0
    rw = nb // nw
    C = 4
    nch = rw // C
    mesh = plsc.VectorSubcoreMesh(core_axis_name="c", subcore_axis_name="s")

    @functools.partial(
        pl.kernel,
        out_type=jax.ShapeDtypeStruct((nb * LP, DIM), jnp.float32),
        mesh=mesh,
        scratch_types=[
            pltpu.VMEM((2, C, L), jnp.int32),
            pltpu.VMEM((2, C, L), jnp.int32),
            pltpu.VMEM((2, C * LP, DIM), jnp.float32),
            pltpu.VMEM((2, C * L, DIM), jnp.float32),
            pltpu.SemaphoreType.DMA,
            pltpu.SemaphoreType.DMA,
            pltpu.SemaphoreType.DMA,
            pltpu.SemaphoreType.DMA,
            pltpu.SemaphoreType.DMA,
            pltpu.SemaphoreType.DMA,
        ],
    )
    def k(idx0_hbm, idx1_hbm, e0_hbm, e1_hbm, out_hbm,
          i0v, i1v, rows, r1, si0, si1, sg0, sg1, st0, st1):
        si = (si0, si1)
        sg = (sg0, sg1)
        st = (st0, st1)
        wid = lax.axis_index("s") * info.num_cores + lax.axis_index("c")
        base = wid * rw

        def idx_copies(ch, buf):
            rb = base + ch * C
            return (
                pltpu.make_async_copy(idx0_hbm.at[pl.ds(rb, C)], i0v.at[buf], si[buf]),
                pltpu.make_async_copy(idx1_hbm.at[pl.ds(rb, C)], i1v.at[buf], si[buf]),
            )

        def gather_copies(buf):
            cps = []
            for c in range(C):
                cps.append(pltpu.make_async_copy(
                    e0_hbm.at[i0v.at[buf, c]],
                    rows.at[buf, pl.ds(c * LP, L)], sg[buf]))
                cps.append(pltpu.make_async_copy(
                    e1_hbm.at[i1v.at[buf, c]],
                    r1.at[buf, pl.ds(c * L, L)], sg[buf]))
            return cps

        def store_copy(ch, buf):
            rb = base + ch * C
            return pltpu.make_async_copy(
                rows.at[buf], out_hbm.at[pl.ds(rb * LP, C * LP)], st[buf])

        def add_chunk(buf):
            for c in range(C):
                def add_row(r, _):
                    for j in range(DIM // NLANE):
                        sl = pl.ds(j * NLANE, NLANE)
                        rows[buf, c * LP + r, sl] = (
                            rows[buf, c * LP + r, sl] + r1[buf, c * L + r, sl])
                    return ()
                lax.fori_loop(0, L, add_row, (), unroll=False)

        z = jnp.zeros((NLANE,), jnp.float32)

        def zero_pad(i, _):
            buf = i // (C * (LP - L))
            rem = i % (C * (LP - L))
            row = (rem // (LP - L)) * LP + L + rem % (LP - L)
            for j in range(DIM // NLANE):
                rows[buf, row, pl.ds(j * NLANE, NLANE)] = z
            return ()

        lax.fori_loop(0, 2 * C * (LP - L), zero_pad, (), unroll=False)

        for cp in idx_copies(0, 0):
            cp.start()
        for cp in idx_copies(0, 0):
            cp.wait()
        for cp in gather_copies(0):
            cp.start()
        for cp in idx_copies(1, 1):
            cp.start()

        def do_chunk(ch, buf):
            ob = 1 - buf

            @pl.when(ch + 1 < nch)
            def _():
                for cp in idx_copies(0, ob):
                    cp.wait()

                @pl.when(ch >= 1)
                def _():
                    store_copy(0, ob).wait()
                for cp in gather_copies(ob):
                    cp.start()

            for cp in gather_copies(buf):
                cp.wait()

            @pl.when(ch + 2 < nch)
            def _():
                for cp in idx_copies(ch + 2, buf):
                    cp.start()

            add_chunk(buf)
            store_copy(ch, buf).start()

        def pair_body(i, _):
            do_chunk(2 * i, 0)
            do_chunk(2 * i + 1, 1)
            return ()

        lax.fori_loop(0, nch // 2, pair_body, (), unroll=False)
        store_copy(0, 0).wait()
        store_copy(0, 1).wait()

    return k(idx0, idx1, emb0, emb1)


def _attn_body(x_ref, s3_ref, w_ref, v_ref, o_ref):
    s3 = s3_ref[...]
    bb = s3.shape[0]
    s2 = s3.reshape(bb * LP, DIM)
    h2 = jnp.tanh(jnp.dot(s2, w_ref[...], preferred_element_type=jnp.float32))
    h3 = h2.reshape(bb, LP, DIM)
    v3 = v_ref[...].reshape(1, 1, DIM)
    sc = jnp.sum(h3 * v3, axis=2)
    xb = x_ref[...]
    lane = lax.broadcasted_iota(jnp.int32, xb.shape, 1)
    length = jnp.sum(((lane >= L) & (xb != 0)).astype(jnp.int32),
                     axis=1, keepdims=True)
    li = lax.broadcasted_iota(jnp.int32, (bb, LP), 1)
    sc = jnp.where(li < length, sc, jnp.float32(-1e9))
    sc = jnp.where(li < L, sc, -jnp.inf)
    m = jnp.max(sc, axis=1, keepdims=True)
    e = jnp.exp(sc - m)
    att = e / jnp.sum(e, axis=1, keepdims=True)
    o_ref[...] = lax.dot_general(
        att, s3, (((1,), (1,)), ((0,), (0,))),
        preferred_element_type=jnp.float32)


def _attn_pool(x, summed3, W_att, v2, nb, row0):
    Bb = 128
    grid = (nb // Bb,)
    blk0 = row0 // Bb
    return pl.pallas_call(
        _attn_body,
        grid=grid,
        in_specs=[
            pl.BlockSpec((Bb, 2 * L), lambda i: (i + blk0, 0)),
            pl.BlockSpec((Bb, LP, DIM), lambda i: (i, 0, 0)),
            pl.BlockSpec((DIM, DIM), lambda i: (0, 0)),
            pl.BlockSpec((1, DIM), lambda i: (0, 0)),
        ],
        out_specs=pl.BlockSpec((Bb, DIM), lambda i: (i, 0)),
        out_shape=jax.ShapeDtypeStruct((nb, DIM), jnp.float32),
    )(x, summed3, W_att, v2)


SPLITS = ((0, 2048), (2048, 1536), (3584, 512))


def kernel(x, emb0, emb1, W_att, v_att):
    idx0 = x[:, :L]
    idx1 = x[:, L:]
    v2 = v_att.reshape(1, DIM)
    summed = [
        _gather_sum(idx0[r0:r0 + nb], idx1[r0:r0 + nb], emb0, emb1)
        for r0, nb in SPLITS
    ]
    outs = [
        _attn_pool(x, summed[k].reshape(nb, LP, DIM), W_att, v2, nb, r0)
        for k, (r0, nb) in enumerate(SPLITS)
    ]
    return jnp.concatenate(outs, axis=0)


# speedup vs baseline: 1.0334x; 1.0334x over previous
import functools

import jax
import jax.numpy as jnp
from jax import lax
from jax.experimental import pallas as pl
from jax.experimental.pallas import tpu as pltpu
from jax.experimental.pallas import tpu_sc as plsc

B = 4096
L = 50
LP = 56
DIM = 128
NLANE = 16


def _gather_sum(idx0, idx1, emb0, emb1):
    nb = idx0.shape[0]
    info = plsc.get_sparse_core_info()
    nw = info.num_cores * info.num_subcores
    assert nb % nw == 0
    rw = nb // nw
    C = 4
    nch = rw // C
    mesh = plsc.VectorSubcoreMesh(core_axis_name="c", subcore_axis_name="s")

    @functools.partial(
        pl.kernel,
        out_type=jax.ShapeDtypeStruct((nb * LP, DIM), jnp.float32),
        mesh=mesh,
        scratch_types=[
            pltpu.VMEM((2, C, L), jnp.int32),
            pltpu.VMEM((2, C, L), jnp.int32),
            pltpu.VMEM((2, C * LP, DIM), jnp.float32),
            pltpu.VMEM((2, C * L, DIM), jnp.float32),
            pltpu.SemaphoreType.DMA,
            pltpu.SemaphoreType.DMA,
            pltpu.SemaphoreType.DMA,
            pltpu.SemaphoreType.DMA,
            pltpu.SemaphoreType.DMA,
            pltpu.SemaphoreType.DMA,
        ],
    )
    def k(idx0_hbm, idx1_hbm, e0_hbm, e1_hbm, out_hbm,
          i0v, i1v, rows, r1, si0, si1, sg0, sg1, st0, st1):
        si = (si0, si1)
        sg = (sg0, sg1)
        st = (st0, st1)
        wid = lax.axis_index("s") * info.num_cores + lax.axis_index("c")
        base = wid * rw

        def idx_copies(ch, buf):
            rb = base + ch * C
            return (
                pltpu.make_async_copy(idx0_hbm.at[pl.ds(rb, C)], i0v.at[buf], si[buf]),
                pltpu.make_async_copy(idx1_hbm.at[pl.ds(rb, C)], i1v.at[buf], si[buf]),
            )

        def gather_copies(buf):
            cps = []
            for c in range(C):
                cps.append(pltpu.make_async_copy(
                    e0_hbm.at[i0v.at[buf, c]],
                    rows.at[buf, pl.ds(c * LP, L)], sg[buf]))
                cps.append(pltpu.make_async_copy(
                    e1_hbm.at[i1v.at[buf, c]],
                    r1.at[buf, pl.ds(c * L, L)], sg[buf]))
            return cps

        def store_copy(ch, buf):
            rb = base + ch * C
            return pltpu.make_async_copy(
                rows.at[buf], out_hbm.at[pl.ds(rb * LP, C * LP)], st[buf])

        def add_chunk(buf):
            for c in range(C):
                def add_row(r, _):
                    for j in range(DIM // NLANE):
                        sl = pl.ds(j * NLANE, NLANE)
                        rows[buf, c * LP + r, sl] = (
                            rows[buf, c * LP + r, sl] + r1[buf, c * L + r, sl])
                    return ()
                lax.fori_loop(0, L, add_row, (), unroll=False)

        z = jnp.zeros((NLANE,), jnp.float32)

        def zero_pad(i, _):
            buf = i // (C * (LP - L))
            rem = i % (C * (LP - L))
            row = (rem // (LP - L)) * LP + L + rem % (LP - L)
            for j in range(DIM // NLANE):
                rows[buf, row, pl.ds(j * NLANE, NLANE)] = z
            return ()

        lax.fori_loop(0, 2 * C * (LP - L), zero_pad, (), unroll=False)

        for cp in idx_copies(0, 0):
            cp.start()
        for cp in idx_copies(0, 0):
            cp.wait()
        for cp in gather_copies(0):
            cp.start()
        for cp in idx_copies(1, 1):
            cp.start()

        def do_chunk(ch, buf):
            ob = 1 - buf

            @pl.when(ch + 1 < nch)
            def _():
                for cp in idx_copies(0, ob):
                    cp.wait()

                @pl.when(ch >= 1)
                def _():
                    store_copy(0, ob).wait()
                for cp in gather_copies(ob):
                    cp.start()

            for cp in gather_copies(buf):
                cp.wait()

            @pl.when(ch + 2 < nch)
            def _():
                for cp in idx_copies(ch + 2, buf):
                    cp.start()

            add_chunk(buf)
            store_copy(ch, buf).start()

        def pair_body(i, _):
            do_chunk(2 * i, 0)
            do_chunk(2 * i + 1, 1)
            return ()

        lax.fori_loop(0, nch // 2, pair_body, (), unroll=False)
        store_copy(0, 0).wait()
        store_copy(0, 1).wait()

    return k(idx0, idx1, emb0, emb1)


def _attn_body(x_ref, s3_ref, w_ref, v_ref, o_ref):
    s3 = s3_ref[...]
    bb = s3.shape[0]
    s2 = s3.reshape(bb * LP, DIM)
    h2 = jnp.tanh(jnp.dot(s2, w_ref[...], preferred_element_type=jnp.float32))
    h3 = h2.reshape(bb, LP, DIM)
    v3 = v_ref[...].reshape(1, 1, DIM)
    sc = jnp.sum(h3 * v3, axis=2)
    xb = x_ref[...]
    lane = lax.broadcasted_iota(jnp.int32, xb.shape, 1)
    length = jnp.sum(((lane >= L) & (xb != 0)).astype(jnp.int32),
                     axis=1, keepdims=True)
    li = lax.broadcasted_iota(jnp.int32, (bb, LP), 1)
    sc = jnp.where(li < length, sc, jnp.float32(-1e9))
    sc = jnp.where(li < L, sc, -jnp.inf)
    m = jnp.max(sc, axis=1, keepdims=True)
    e = jnp.exp(sc - m)
    att = e / jnp.sum(e, axis=1, keepdims=True)
    o_ref[...] = lax.dot_general(
        att, s3, (((1,), (1,)), ((0,), (0,))),
        preferred_element_type=jnp.float32)


def _attn_pool(x, summed3, W_att, v2, nb, row0):
    Bb = 256
    grid = (nb // Bb,)
    blk0 = row0 // Bb
    return pl.pallas_call(
        _attn_body,
        grid=grid,
        in_specs=[
            pl.BlockSpec((Bb, 2 * L), lambda i: (i + blk0, 0)),
            pl.BlockSpec((Bb, LP, DIM), lambda i: (i, 0, 0)),
            pl.BlockSpec((DIM, DIM), lambda i: (0, 0)),
            pl.BlockSpec((1, DIM), lambda i: (0, 0)),
        ],
        out_specs=pl.BlockSpec((Bb, DIM), lambda i: (i, 0)),
        out_shape=jax.ShapeDtypeStruct((nb, DIM), jnp.float32),
    )(x, summed3, W_att, v2)


SPLITS = ((0, 2816), (2816, 1280))


def kernel(x, emb0, emb1, W_att, v_att):
    idx0 = x[:, :L]
    idx1 = x[:, L:]
    v2 = v_att.reshape(1, DIM)
    summed = [
        _gather_sum(idx0[r0:r0 + nb], idx1[r0:r0 + nb], emb0, emb1)
        for r0, nb in SPLITS
    ]
    outs = [
        _attn_pool(x, summed[k].reshape(nb, LP, DIM), W_att, v2, nb, r0)
        for k, (r0, nb) in enumerate(SPLITS)
    ]
    return jnp.concatenate(outs, axis=0)
